# speedup vs baseline: 1.0997x; 1.0997x over previous
import jax
import jax.numpy as jnp
from jax.experimental import pallas as pl
from jax.experimental.pallas import tpu as pltpu

_BH = 512


def _body(x_ref, y_ref, X_ref, y_blk, o_blk):
    i = pl.program_id(0)
    j = pl.program_id(1)
    o_blk[...] = y_blk[...]
    r_loc = y_ref[i] - j * _BH
    c = x_ref[i]

    @pl.when((r_loc >= 0) & (r_loc < _BH))
    def _inject():
        row = y_blk[0, pl.ds(r_loc, 1), :]
        w = row.shape[-1]
        colmask = jax.lax.broadcasted_iota(jnp.int32, (1, w), 1) == c
        o_blk[0, pl.ds(r_loc, 1), :] = row + jnp.where(colmask, X_ref[0], 0.0)


def kernel(Y, X, x, y):
    n, h, w = Y.shape
    grid = (n, h // _BH)
    return pl.pallas_call(
        _body,
        grid=grid,
        in_specs=[
            pl.BlockSpec(memory_space=pltpu.SMEM),
            pl.BlockSpec(memory_space=pltpu.SMEM),
            pl.BlockSpec(memory_space=pltpu.SMEM),
            pl.BlockSpec((1, _BH, w), lambda i, j: (i, j, 0)),
        ],
        out_specs=pl.BlockSpec((1, _BH, w), lambda i, j: (i, j, 0)),
        out_shape=jax.ShapeDtypeStruct(Y.shape, Y.dtype),
        compiler_params=pltpu.CompilerParams(
            dimension_semantics=("parallel", "parallel"),
        ),
    )(x, y, X, Y)


# speedup vs baseline: 1.1124x; 1.0116x over previous
import jax
import jax.numpy as jnp
from jax.experimental import pallas as pl
from jax.experimental.pallas import tpu as pltpu

_BH = 1024


def _body(x_ref, y_ref, X_ref, y_blk, o_blk):
    i = pl.program_id(0)
    j = pl.program_id(1)
    o_blk[...] = y_blk[...]
    r_loc = y_ref[i] - j * _BH
    c = x_ref[i]

    @pl.when((r_loc >= 0) & (r_loc < _BH))
    def _inject():
        row = y_blk[0, pl.ds(r_loc, 1), :]
        w = row.shape[-1]
        colmask = jax.lax.broadcasted_iota(jnp.int32, (1, w), 1) == c
        o_blk[0, pl.ds(r_loc, 1), :] = row + jnp.where(colmask, X_ref[0], 0.0)


def kernel(Y, X, x, y):
    n, h, w = Y.shape
    grid = (n, h // _BH)
    return pl.pallas_call(
        _body,
        grid=grid,
        in_specs=[
            pl.BlockSpec(memory_space=pltpu.SMEM),
            pl.BlockSpec(memory_space=pltpu.SMEM),
            pl.BlockSpec(memory_space=pltpu.SMEM),
            pl.BlockSpec((1, _BH, w), lambda i, j: (i, j, 0)),
        ],
        out_specs=pl.BlockSpec((1, _BH, w), lambda i, j: (i, j, 0)),
        out_shape=jax.ShapeDtypeStruct(Y.shape, Y.dtype),
        compiler_params=pltpu.CompilerParams(
            dimension_semantics=("parallel", "parallel"),
        ),
    )(x, y, X, Y)
